# Initial kernel scaffold; baseline (speedup 1.0000x reference)
import jax
import jax.numpy as jnp
from jax.experimental import pallas as pl


def kernel(image_features, depths, c2w_cond, intrinsic_cond):
    raise NotImplementedError("write your pallas kernel here")


# speedup vs baseline: 1.1293x; 1.1293x over previous
import jax
import jax.numpy as jnp
from jax import lax
from jax.experimental import pallas as pl
from jax.experimental.pallas import tpu as pltpu
from jax.experimental.pallas import tpu_sc as plsc

G = 128
G2 = G * G
NPLANES = 3
FLAT = NPLANES * G2
DUMP = FLAT
CH = 2048
L = 16
HW = 65536
NV = 4
NC = 96


def _read_consts(cbuf):
    va = cbuf[pl.ds(0, 16)]
    vb = cbuf[pl.ds(16, 16)]
    vals = [va[i] for i in range(16)] + [vb[i] for i in range(8)]
    return vals[0:9], vals[9:18], vals[18:21]


def _bf16_round(x):
    bits = plsc.bitcast(x, jnp.int32)
    rounded = (bits + 0x8000 + ((bits >> 16) & 1)) & jnp.int32(-65536)
    return plsc.bitcast(rounded, jnp.float32)


def _project16(k, r, t, u, vr, dd):
    cx = k[0] * u + k[1] * vr + k[2]
    cy = k[3] * u + k[4] * vr + k[5]
    cz = k[6] * u + k[7] * vr + k[8]
    px = _bf16_round(cx * dd)
    py = _bf16_round(cy * dd)
    pz = _bf16_round(cz * dd)
    wx = r[0] * px + r[1] * py + r[2] * pz + t[0]
    wy = r[3] * px + r[4] * py + r[5] * pz + t[1]
    wz = r[6] * px + r[7] * py + r[8] * pz + t[2]
    return wx, wy, wz


def _bounds_kernel(depths_hbm, consts_hbm, out_hbm, fbuf, cbuf, vbuf):
    c = lax.axis_index("c")
    s = lax.axis_index("s")
    wid = c * 16 + s
    vi = wid // 4
    qoff = (wid % 4) * 16384

    pltpu.sync_copy(consts_hbm.at[pl.ds(vi * 32, 32)], cbuf)
    k, r, t = _read_consts(cbuf)
    lanes = lax.iota(jnp.int32, 16).astype(jnp.float32)

    inf = jnp.full((L,), jnp.inf, jnp.float32)
    for i in range(3):
        vbuf[pl.ds(i * 16, L)] = inf
        vbuf[pl.ds((3 + i) * 16, L)] = -inf

    def chunk(ci, _):
        off = qoff + ci * CH
        pltpu.sync_copy(depths_hbm.at[pl.ds(vi * HW + off, CH)], fbuf)
        row0 = off // 256

        def jbody(j, carry):
            mnx, mny, mnz, mxx, mxy, mxz = carry
            u0 = (j % 16) * 16
            vr = (row0 + j // 16).astype(jnp.float32)
            u = u0.astype(jnp.float32) + lanes
            d = fbuf[pl.ds(j * 16, L)]
            mask = d != 0.0
            dd = jnp.where(mask, d + 0.01, d)
            wx, wy, wz = _project16(k, r, t, u, vr, dd)
            mnx = jnp.minimum(mnx, jnp.where(mask, wx, jnp.inf))
            mny = jnp.minimum(mny, jnp.where(mask, wy, jnp.inf))
            mnz = jnp.minimum(mnz, jnp.where(mask, wz, jnp.inf))
            mxx = jnp.maximum(mxx, jnp.where(mask, wx, -jnp.inf))
            mxy = jnp.maximum(mxy, jnp.where(mask, wy, -jnp.inf))
            mxz = jnp.maximum(mxz, jnp.where(mask, wz, -jnp.inf))
            return mnx, mny, mnz, mxx, mxy, mxz

        init = tuple(vbuf[pl.ds(i * 16, L)] for i in range(6))
        res = lax.fori_loop(0, CH // L, jbody, init)
        for i in range(6):
            vbuf[pl.ds(i * 16, L)] = res[i]
        return 0

    lax.fori_loop(0, 16384 // CH, chunk, 0)
    pltpu.sync_copy(vbuf, out_hbm.at[pl.ds(wid * 96, 96)])


def _scatter_kernel(feats_hbm, depths_hbm, consts_hbm, sb_hbm, out_hbm,
                    sums, fbuf, ibuf, rbuf, obuf, tbuf, cbuf, sbuf,
                    idx_sp, recip_sp, trec_sp):
    c = lax.axis_index("c")
    s = lax.axis_index("s")
    b = c
    lanes = lax.iota(jnp.int32, 16).astype(jnp.float32)
    zero16 = jnp.zeros((L,), jnp.float32)
    ones16 = jnp.ones((L,), jnp.float32)

    v1 = s // 4
    qoff = (s % 4) * 16384
    pltpu.sync_copy(sb_hbm, sbuf)
    sbv = sbuf[pl.ds(0, 16)]
    sb = [sbv[i] for i in range(6)]
    den_x = sb[1] - sb[0]
    den_y = sb[3] - sb[2]
    den_z = sb[5] - sb[4]
    pltpu.sync_copy(consts_hbm.at[pl.ds((b * NV + v1) * 32, 32)], cbuf)
    k, r, t = _read_consts(cbuf)

    def b1_chunk(ci, _):
        off = qoff + ci * CH
        pltpu.sync_copy(depths_hbm.at[pl.ds((b * NV + v1) * HW + off, CH)],
                        fbuf)
        row0 = off // 256

        def jbody(j, _):
            u0 = (j % 16) * 16
            vr = (row0 + j // 16).astype(jnp.float32)
            u = u0.astype(jnp.float32) + lanes
            d = fbuf[pl.ds(j * 16, L)]
            mask = d != 0.0
            dd = jnp.where(mask, d + 0.02, d)
            wx, wy, wz = _project16(k, r, t, u, vr, dd)
            nx = 2.0 * (wx - sb[0]) / den_x - 1.0
            ny = 2.0 * (wy - sb[2]) / den_y - 1.0
            nz = 2.0 * (wz - sb[4]) / den_z - 1.0
            cxi = jnp.clip(((nx * 0.5 + 0.5) * (G - 1)).astype(jnp.int32), 0, G - 1)
            cyi = jnp.clip(((ny * 0.5 + 0.5) * (G - 1)).astype(jnp.int32), 0, G - 1)
            czi = jnp.clip(((nz * 0.5 + 0.5) * (G - 1)).astype(jnp.int32), 0, G - 1)
            dump = jnp.full((L,), DUMP, jnp.int32)
            ibuf[pl.ds(j * 16, L)] = jnp.where(mask, cxi * G + cyi, dump)
            ibuf[pl.ds(CH + j * 16, L)] = jnp.where(
                mask, cxi * G + czi + G2, dump)
            ibuf[pl.ds(2 * CH + j * 16, L)] = jnp.where(
                mask, cyi * G + czi + 2 * G2, dump)
            return 0

        lax.fori_loop(0, CH // L, jbody, 0)
        for p in range(NPLANES):
            pltpu.sync_copy(
                ibuf.at[pl.ds(p * CH, CH)],
                idx_sp.at[pl.ds((v1 * NPLANES + p) * HW + off, CH)])
        return 0

    lax.fori_loop(0, 16384 // CH, b1_chunk, 0)
    plsc.subcore_barrier()

    @pl.when(s < 4)
    def _b2():
        v = s

        def zbody(i, _):
            sums[pl.ds(i * 16, L)] = zero16
            return 0

        lax.fori_loop(0, (FLAT + L) // L, zbody, 0)

        def cchunk(ci, _):
            off = ci * CH
            for p in range(NPLANES):
                pltpu.sync_copy(
                    idx_sp.at[pl.ds((v * NPLANES + p) * HW + off, CH)],
                    ibuf.at[pl.ds(p * CH, CH)])

            def jbody(j, _):
                for p in range(NPLANES):
                    iv = ibuf[pl.ds(p * CH + j * 16, L)]
                    plsc.addupdate_scatter(sums, [iv], ones16)
                return 0

            lax.fori_loop(0, CH // L, jbody, 0)
            return 0

        lax.fori_loop(0, HW // CH, cchunk, 0)

        def rbody(i, _):
            cnt = sums[pl.ds(i * 16, L)]
            sums[pl.ds(i * 16, L)] = jnp.where(cnt > 0.0, 1.0 / cnt, 0.0)
            return 0

        lax.fori_loop(0, FLAT // L, rbody, 0)
        pltpu.sync_copy(sums.at[pl.ds(0, FLAT)],
                        recip_sp.at[pl.ds(v * FLAT, FLAT)])

    plsc.subcore_barrier()

    span = FLAT // 32
    for sub in range(2):
        base = s * (FLAT // 16) + sub * span

        def ztbody(i, _):
            tbuf[pl.ds(i * 16, L)] = zero16
            return 0

        lax.fori_loop(0, span // L, ztbody, 0)
        for v in range(NV):
            pltpu.sync_copy(recip_sp.at[pl.ds(v * FLAT + base, span)],
                            rbuf.at[pl.ds(0, span)])

            def tbody(i, _):
                rv = rbuf[pl.ds(i * 16, L)]
                tbuf[pl.ds(i * 16, L)] += jnp.where(rv > 0.0, 1.0, 0.0)
                return 0

            lax.fori_loop(0, span // L, tbody, 0)

        def trbody(i, _):
            tv = tbuf[pl.ds(i * 16, L)]
            tbuf[pl.ds(i * 16, L)] = 1.0 / jnp.maximum(tv, 1e-6)
            return 0

        lax.fori_loop(0, span // L, trbody, 0)
        pltpu.sync_copy(tbuf.at[pl.ds(0, span)], trec_sp.at[pl.ds(base, span)])
    plsc.subcore_barrier()

    for kch in range(6):
        ch = kch * 16 + s

        for v in range(NV):
            def zsbody(i, _):
                sums[pl.ds(i * 16, L)] = zero16
                return 0

            lax.fori_loop(0, (FLAT + L) // L, zsbody, 0)

            def schunk(ci, _):
                off = ci * CH
                pltpu.sync_copy(
                    feats_hbm.at[pl.ds(((b * NV + v) * NC + ch) * HW + off,
                                       CH)], fbuf)
                for p in range(NPLANES):
                    pltpu.sync_copy(
                        idx_sp.at[pl.ds((v * NPLANES + p) * HW + off, CH)],
                        ibuf.at[pl.ds(p * CH, CH)])

                def jbody(j, _):
                    fv = fbuf[pl.ds(j * 16, L)]
                    for p in range(NPLANES):
                        iv = ibuf[pl.ds(p * CH + j * 16, L)]
                        plsc.addupdate_scatter(sums, [iv], fv)
                    return 0

                lax.fori_loop(0, CH // L, jbody, 0)
                return 0

            lax.fori_loop(0, HW // CH, schunk, 0)

            for p in range(NPLANES):
                def mchunk(gi, _):
                    goff = p * G2 + gi * CH
                    ooff = ((b * NPLANES + p) * NC + ch) * G2 + gi * CH
                    pltpu.sync_copy(recip_sp.at[pl.ds(v * FLAT + goff, CH)],
                                    rbuf.at[pl.ds(0, CH)])
                    if v > 0:
                        pltpu.sync_copy(out_hbm.at[pl.ds(ooff, CH)], obuf)
                    if v == NV - 1:
                        pltpu.sync_copy(trec_sp.at[pl.ds(goff, CH)],
                                        tbuf.at[pl.ds(0, CH)])

                    def jbody(j, _):
                        sl16 = pl.ds(j * 16, L)
                        mean = sums[pl.ds(goff + j * 16, L)] * rbuf[sl16]
                        if v == 0:
                            acc = mean
                        else:
                            acc = obuf[sl16] + mean
                        if v == NV - 1:
                            acc = acc * tbuf[sl16]
                        obuf[sl16] = acc
                        return 0

                    lax.fori_loop(0, CH // L, jbody, 0)
                    pltpu.sync_copy(obuf, out_hbm.at[pl.ds(ooff, CH)])
                    return 0

                lax.fori_loop(0, G2 // CH, mchunk, 0)


def kernel(image_features, depths, c2w_cond, intrinsic_cond):
    B, Nv, C, H, W = image_features.shape
    feats_r = image_features.reshape(-1)
    depths_r = depths.reshape(-1)

    k_inv = jnp.linalg.inv(intrinsic_cond)
    rot = c2w_cond[:, :, :3, :3]
    trans = c2w_cond[:, :, :3, 3]
    consts = jnp.concatenate(
        [k_inv.reshape(B, Nv, 9), rot.reshape(B, Nv, 9), trans,
         jnp.zeros((B, Nv, 11), jnp.float32)], axis=-1).reshape(-1)
    consts = consts.astype(jnp.bfloat16).astype(jnp.float32)

    mesh = plsc.VectorSubcoreMesh(core_axis_name="c", subcore_axis_name="s",
                                  num_cores=2, num_subcores=16)

    bounds_call = pl.kernel(
        _bounds_kernel, mesh=mesh,
        compiler_params=pltpu.CompilerParams(needs_layout_passes=False),
        out_type=jax.ShapeDtypeStruct((32 * 96,), jnp.float32),
        scratch_types=[
            pltpu.VMEM((CH,), jnp.float32),
            pltpu.VMEM((32,), jnp.float32),
            pltpu.VMEM((96,), jnp.float32),
        ])
    parts = bounds_call(depths_r, consts).reshape(32, 6, 16)

    mins = jnp.minimum(parts[:, :3].min(axis=(0, 2)), 0.0)
    maxs = jnp.maximum(parts[:, 3:].max(axis=(0, 2)), 0.0)
    b0, b2_, b4 = mins[0], mins[1], mins[2]
    b1, b3, b5 = maxs[0], maxs[1], maxs[2]
    pad = 0.05
    sb = (b0 - pad * (b1 - b0), b1 + pad * (b1 - b0),
          b2_ - pad * (b3 - b2_), b3 + pad * (b3 - b2_),
          b4 - pad * (b5 - b4), b5 + pad * (b5 - b4))
    sb_arr = jnp.concatenate([jnp.stack(sb), jnp.zeros((10,), jnp.float32)])

    scatter_call = pl.kernel(
        _scatter_kernel, mesh=mesh,
        compiler_params=pltpu.CompilerParams(needs_layout_passes=False),
        out_type=jax.ShapeDtypeStruct((B * NPLANES * C * G2,), jnp.float32),
        scratch_types=[
            pltpu.VMEM((FLAT + L,), jnp.float32),
            pltpu.VMEM((CH,), jnp.float32),
            pltpu.VMEM((NPLANES * CH,), jnp.int32),
            pltpu.VMEM((CH,), jnp.float32),
            pltpu.VMEM((CH,), jnp.float32),
            pltpu.VMEM((CH,), jnp.float32),
            pltpu.VMEM((32,), jnp.float32),
            pltpu.VMEM((16,), jnp.float32),
            pltpu.VMEM_SHARED((Nv * NPLANES * HW,), jnp.int32),
            pltpu.VMEM_SHARED((Nv * FLAT,), jnp.float32),
            pltpu.VMEM_SHARED((FLAT,), jnp.float32),
        ])
    out_flat = scatter_call(feats_r, depths_r, consts, sb_arr)
    out = out_flat.reshape(B, NPLANES, C, G, G)
    return out, sb


# speedup vs baseline: 1.3735x; 1.2163x over previous
import jax
import jax.numpy as jnp
from jax import lax
from jax.experimental import pallas as pl
from jax.experimental.pallas import tpu as pltpu
from jax.experimental.pallas import tpu_sc as plsc

G = 128
G2 = G * G
NPLANES = 3
FLAT = NPLANES * G2
DUMP = FLAT
CHB = 1024
CHM = 1024
L = 16
HW = 65536
NV = 4
NC = 96
NCHUNK = HW // CHB
NMCH = FLAT // CHM


def _read_consts(cbuf):
    va = cbuf[pl.ds(0, 16)]
    vb = cbuf[pl.ds(16, 16)]
    vals = [va[i] for i in range(16)] + [vb[i] for i in range(8)]
    return vals[0:9], vals[9:18], vals[18:21]


def _bf16_round(x):
    bits = plsc.bitcast(x, jnp.int32)
    rounded = (bits + 0x8000 + ((bits >> 16) & 1)) & jnp.int32(-65536)
    return plsc.bitcast(rounded, jnp.float32)


def _project16(k, r, t, u, vr, dd):
    cx = k[0] * u + k[1] * vr + k[2]
    cy = k[3] * u + k[4] * vr + k[5]
    cz = k[6] * u + k[7] * vr + k[8]
    px = _bf16_round(cx * dd)
    py = _bf16_round(cy * dd)
    pz = _bf16_round(cz * dd)
    wx = r[0] * px + r[1] * py + r[2] * pz + t[0]
    wy = r[3] * px + r[4] * py + r[5] * pz + t[1]
    wz = r[6] * px + r[7] * py + r[8] * pz + t[2]
    return wx, wy, wz


def _bounds_kernel(depths_hbm, consts_hbm, out_hbm, fbuf, cbuf, vbuf):
    c = lax.axis_index("c")
    s = lax.axis_index("s")
    wid = c * 16 + s
    vi = wid // 4
    qoff = (wid % 4) * 16384

    pltpu.sync_copy(consts_hbm.at[pl.ds(vi * 32, 32)], cbuf)
    k, r, t = _read_consts(cbuf)
    lanes = lax.iota(jnp.int32, 16).astype(jnp.float32)

    inf = jnp.full((L,), jnp.inf, jnp.float32)
    for i in range(3):
        vbuf[pl.ds(i * 16, L)] = inf
        vbuf[pl.ds((3 + i) * 16, L)] = -inf

    def chunk(ci, _):
        off = qoff + ci * CHB
        pltpu.sync_copy(depths_hbm.at[pl.ds(vi * HW + off, CHB)],
                        fbuf.at[pl.ds(0, CHB)])
        row0 = off // 256

        def jbody(j, carry):
            mnx, mny, mnz, mxx, mxy, mxz = carry
            u0 = (j % 16) * 16
            vr = (row0 + j // 16).astype(jnp.float32)
            u = u0.astype(jnp.float32) + lanes
            d = fbuf[pl.ds(j * 16, L)]
            mask = d != 0.0
            dd = jnp.where(mask, d + 0.01, d)
            wx, wy, wz = _project16(k, r, t, u, vr, dd)
            mnx = jnp.minimum(mnx, jnp.where(mask, wx, jnp.inf))
            mny = jnp.minimum(mny, jnp.where(mask, wy, jnp.inf))
            mnz = jnp.minimum(mnz, jnp.where(mask, wz, jnp.inf))
            mxx = jnp.maximum(mxx, jnp.where(mask, wx, -jnp.inf))
            mxy = jnp.maximum(mxy, jnp.where(mask, wy, -jnp.inf))
            mxz = jnp.maximum(mxz, jnp.where(mask, wz, -jnp.inf))
            return mnx, mny, mnz, mxx, mxy, mxz

        init = tuple(vbuf[pl.ds(i * 16, L)] for i in range(6))
        res = lax.fori_loop(0, CHB // L, jbody, init)
        for i in range(6):
            vbuf[pl.ds(i * 16, L)] = res[i]
        return 0

    lax.fori_loop(0, 16384 // CHB, chunk, 0)
    pltpu.sync_copy(vbuf, out_hbm.at[pl.ds(wid * 96, 96)])


def _scatter_kernel(feats_hbm, depths_hbm, consts_hbm, sb_hbm, out_hbm,
                    sums, fbuf, ibuf, rbuf, obuf, tbuf, cbuf, sbuf,
                    sf0, sf1, si0, si1, sr0, sr1, so0, so1, st0, st1,
                    sw0, sw1, idx_sp, recip_sp, trec_sp):
    c = lax.axis_index("c")
    s = lax.axis_index("s")
    b = c
    lanes = lax.iota(jnp.int32, 16).astype(jnp.float32)
    zero16 = jnp.zeros((L,), jnp.float32)
    ones16 = jnp.ones((L,), jnp.float32)
    semf = (sf0, sf1)
    semi = (si0, si1)
    semr = (sr0, sr1)
    semo = (so0, so1)
    semt = (st0, st1)
    semw = (sw0, sw1)

    v1 = s // 4
    qoff = (s % 4) * 16384
    pltpu.sync_copy(sb_hbm, sbuf)
    sbv = sbuf[pl.ds(0, 16)]
    sb = [sbv[i] for i in range(6)]
    den_x = sb[1] - sb[0]
    den_y = sb[3] - sb[2]
    den_z = sb[5] - sb[4]
    pltpu.sync_copy(consts_hbm.at[pl.ds((b * NV + v1) * 32, 32)], cbuf)
    k, r, t = _read_consts(cbuf)

    def b1_chunk(ci, _):
        off = qoff + ci * CHB
        pltpu.sync_copy(depths_hbm.at[pl.ds((b * NV + v1) * HW + off, CHB)],
                        fbuf.at[pl.ds(0, CHB)])
        row0 = off // 256

        def jbody(j, _):
            u0 = (j % 16) * 16
            vr = (row0 + j // 16).astype(jnp.float32)
            u = u0.astype(jnp.float32) + lanes
            d = fbuf[pl.ds(j * 16, L)]
            mask = d != 0.0
            dd = jnp.where(mask, d + 0.02, d)
            wx, wy, wz = _project16(k, r, t, u, vr, dd)
            nx = 2.0 * (wx - sb[0]) / den_x - 1.0
            ny = 2.0 * (wy - sb[2]) / den_y - 1.0
            nz = 2.0 * (wz - sb[4]) / den_z - 1.0
            cxi = jnp.clip(((nx * 0.5 + 0.5) * (G - 1)).astype(jnp.int32), 0, G - 1)
            cyi = jnp.clip(((ny * 0.5 + 0.5) * (G - 1)).astype(jnp.int32), 0, G - 1)
            czi = jnp.clip(((nz * 0.5 + 0.5) * (G - 1)).astype(jnp.int32), 0, G - 1)
            dump = jnp.full((L,), DUMP, jnp.int32)
            ibuf[pl.ds(j * 16, L)] = jnp.where(mask, cxi * G + cyi, dump)
            ibuf[pl.ds(CHB + j * 16, L)] = jnp.where(
                mask, cxi * G + czi + G2, dump)
            ibuf[pl.ds(2 * CHB + j * 16, L)] = jnp.where(
                mask, cyi * G + czi + 2 * G2, dump)
            return 0

        lax.fori_loop(0, CHB // L, jbody, 0)
        g = qoff // CHB + ci
        pltpu.sync_copy(ibuf.at[pl.ds(0, 3 * CHB)],
                        idx_sp.at[pl.ds(v1 * 3 * HW + g * 3 * CHB, 3 * CHB)])
        return 0

    lax.fori_loop(0, 16384 // CHB, b1_chunk, 0)
    plsc.subcore_barrier()

    @pl.when(s < 4)
    def _b2():
        v = s

        def zbody(i, _):
            sums[pl.ds(i * 16, L)] = zero16
            return 0

        lax.fori_loop(0, (FLAT + L) // L, zbody, 0)

        def cchunk(gi, _):
            pltpu.sync_copy(
                idx_sp.at[pl.ds(v * 3 * HW + gi * 3 * CHB, 3 * CHB)],
                ibuf.at[pl.ds(0, 3 * CHB)])

            def jbody(j, _):
                for p in range(NPLANES):
                    iv = ibuf[pl.ds(p * CHB + j * 16, L)]
                    plsc.addupdate_scatter(sums, [iv], ones16)
                return 0

            lax.fori_loop(0, CHB // L, jbody, 0)
            return 0

        lax.fori_loop(0, NCHUNK, cchunk, 0)

        def rbody(i, _):
            cnt = sums[pl.ds(i * 16, L)]
            sums[pl.ds(i * 16, L)] = jnp.where(cnt > 0.0, 1.0 / cnt, 0.0)
            return 0

        lax.fori_loop(0, FLAT // L, rbody, 0)
        pltpu.sync_copy(sums.at[pl.ds(0, FLAT)],
                        recip_sp.at[pl.ds(v * FLAT, FLAT)])

    plsc.subcore_barrier()

    span = 1024
    for sub in range(3):
        base = s * 3072 + sub * span

        def ztbody(i, _):
            tbuf[pl.ds(i * 16, L)] = zero16
            return 0

        lax.fori_loop(0, span // L, ztbody, 0)
        for v in range(NV):
            pltpu.sync_copy(recip_sp.at[pl.ds(v * FLAT + base, span)],
                            rbuf.at[pl.ds(0, span)])

            def tbody(i, _):
                rv = rbuf[pl.ds(i * 16, L)]
                tbuf[pl.ds(i * 16, L)] += jnp.where(rv > 0.0, 1.0, 0.0)
                return 0

            lax.fori_loop(0, span // L, tbody, 0)

        def trbody(i, _):
            tv = tbuf[pl.ds(i * 16, L)]
            tbuf[pl.ds(i * 16, L)] = 1.0 / jnp.maximum(tv, 1e-6)
            return 0

        lax.fori_loop(0, span // L, trbody, 0)
        pltpu.sync_copy(tbuf.at[pl.ds(0, span)], trec_sp.at[pl.ds(base, span)])
    plsc.subcore_barrier()

    for kch in range(6):
        ch = kch * 16 + s

        for v in range(NV):
            fbase = ((b * NV + v) * NC + ch) * HW
            ibase = v * 3 * HW

            def zsbody(i, _):
                sums[pl.ds(i * 16, L)] = zero16
                return 0

            lax.fori_loop(0, (FLAT + L) // L, zsbody, 0)

            def sc_cp(g, sl):
                return (
                    pltpu.make_async_copy(
                        feats_hbm.at[pl.ds(fbase + g * CHB, CHB)],
                        fbuf.at[pl.ds(sl * CHB, CHB)], semf[sl]),
                    pltpu.make_async_copy(
                        idx_sp.at[pl.ds(ibase + g * 3 * CHB, 3 * CHB)],
                        ibuf.at[pl.ds(sl * 3 * CHB, 3 * CHB)], semi[sl]),
                )

            def sc_start(g, sl):
                for cp in sc_cp(g, sl):
                    cp.start()

            def sc_wait(g, sl):
                for cp in sc_cp(g, sl):
                    cp.wait()

            def sc_compute(sl):
                def jbody(j, _):
                    fv = fbuf[pl.ds(sl * CHB + j * 16, L)]
                    for p in range(NPLANES):
                        iv = ibuf[pl.ds(sl * 3 * CHB + p * CHB + j * 16, L)]
                        plsc.addupdate_scatter(sums, [iv], fv)
                    return 0

                lax.fori_loop(0, CHB // L, jbody, 0)

            sc_start(0, 0)

            def sc_pair(ci, _):
                g0 = ci * 2
                g1 = g0 + 1
                gn = jnp.minimum(g0 + 2, NCHUNK - 1)
                sc_start(g1, 1)
                sc_wait(g0, 0)
                sc_compute(0)
                sc_start(gn, 0)
                sc_wait(g1, 1)
                sc_compute(1)
                return 0

            lax.fori_loop(0, NCHUNK // 2, sc_pair, 0)
            sc_wait(NCHUNK - 1, 0)

            def m_ooff(g):
                return ((b * NPLANES + g // 16) * NC + ch) * G2 \
                    + (g % 16) * CHM

            def m_reads(g, sl):
                cps = [pltpu.make_async_copy(
                    recip_sp.at[pl.ds(v * FLAT + g * CHM, CHM)],
                    rbuf.at[pl.ds(sl * CHM, CHM)], semr[sl])]
                if v > 0:
                    cps.append(pltpu.make_async_copy(
                        out_hbm.at[pl.ds(m_ooff(g), CHM)],
                        obuf.at[pl.ds(sl * CHM, CHM)], semo[sl]))
                if v == NV - 1:
                    cps.append(pltpu.make_async_copy(
                        trec_sp.at[pl.ds(g * CHM, CHM)],
                        tbuf.at[pl.ds(sl * CHM, CHM)], semt[sl]))
                return cps

            def m_write(g, sl):
                return pltpu.make_async_copy(
                    obuf.at[pl.ds(sl * CHM, CHM)],
                    out_hbm.at[pl.ds(m_ooff(g), CHM)], semw[sl])

            def m_compute(g, sl):
                def jbody(j, _):
                    sl16 = pl.ds(sl * CHM + j * 16, L)
                    mean = sums[pl.ds(g * CHM + j * 16, L)] * rbuf[sl16]
                    if v == 0:
                        acc = mean
                    else:
                        acc = obuf[sl16] + mean
                    if v == NV - 1:
                        acc = acc * tbuf[sl16]
                    obuf[sl16] = acc
                    return 0

                lax.fori_loop(0, CHM // L, jbody, 0)

            def m_pair(ci, _):
                g0 = ci * 2
                g1 = g0 + 1

                @pl.when(ci > 0)
                def _drain():
                    m_write(g0, 0).wait()
                    m_write(g1, 1).wait()

                for cp in m_reads(g0, 0):
                    cp.start()
                for cp in m_reads(g1, 1):
                    cp.start()
                for cp in m_reads(g0, 0):
                    cp.wait()
                m_compute(g0, 0)
                m_write(g0, 0).start()
                for cp in m_reads(g1, 1):
                    cp.wait()
                m_compute(g1, 1)
                m_write(g1, 1).start()
                return 0

            lax.fori_loop(0, NMCH // 2, m_pair, 0)
            m_write(NMCH - 2, 0).wait()
            m_write(NMCH - 1, 1).wait()


def kernel(image_features, depths, c2w_cond, intrinsic_cond):
    B, Nv, C, H, W = image_features.shape
    feats_r = image_features.reshape(-1)
    depths_r = depths.reshape(-1)

    k_inv = jnp.linalg.inv(intrinsic_cond)
    rot = c2w_cond[:, :, :3, :3]
    trans = c2w_cond[:, :, :3, 3]
    consts = jnp.concatenate(
        [k_inv.reshape(B, Nv, 9), rot.reshape(B, Nv, 9), trans,
         jnp.zeros((B, Nv, 11), jnp.float32)], axis=-1).reshape(-1)
    consts = consts.astype(jnp.bfloat16).astype(jnp.float32)

    mesh = plsc.VectorSubcoreMesh(core_axis_name="c", subcore_axis_name="s",
                                  num_cores=2, num_subcores=16)

    bounds_call = pl.kernel(
        _bounds_kernel, mesh=mesh,
        compiler_params=pltpu.CompilerParams(needs_layout_passes=False),
        out_type=jax.ShapeDtypeStruct((32 * 96,), jnp.float32),
        scratch_types=[
            pltpu.VMEM((CHB,), jnp.float32),
            pltpu.VMEM((32,), jnp.float32),
            pltpu.VMEM((96,), jnp.float32),
        ])
    parts = bounds_call(depths_r, consts).reshape(32, 6, 16)

    mins = jnp.minimum(parts[:, :3].min(axis=(0, 2)), 0.0)
    maxs = jnp.maximum(parts[:, 3:].max(axis=(0, 2)), 0.0)
    b0, b2_, b4 = mins[0], mins[1], mins[2]
    b1, b3, b5 = maxs[0], maxs[1], maxs[2]
    pad = 0.05
    sb = (b0 - pad * (b1 - b0), b1 + pad * (b1 - b0),
          b2_ - pad * (b3 - b2_), b3 + pad * (b3 - b2_),
          b4 - pad * (b5 - b4), b5 + pad * (b5 - b4))
    sb_arr = jnp.concatenate([jnp.stack(sb), jnp.zeros((10,), jnp.float32)])

    scatter_call = pl.kernel(
        _scatter_kernel, mesh=mesh,
        compiler_params=pltpu.CompilerParams(needs_layout_passes=False),
        out_type=jax.ShapeDtypeStruct((B * NPLANES * C * G2,), jnp.float32),
        scratch_types=[
            pltpu.VMEM((FLAT + L,), jnp.float32),
            pltpu.VMEM((2 * CHB,), jnp.float32),
            pltpu.VMEM((2 * 3 * CHB,), jnp.int32),
            pltpu.VMEM((2 * CHM,), jnp.float32),
            pltpu.VMEM((2 * CHM,), jnp.float32),
            pltpu.VMEM((2 * CHM,), jnp.float32),
            pltpu.VMEM((32,), jnp.float32),
            pltpu.VMEM((16,), jnp.float32),
        ] + [pltpu.SemaphoreType.DMA] * 12 + [
            pltpu.VMEM_SHARED((NV * NPLANES * HW,), jnp.int32),
            pltpu.VMEM_SHARED((NV * FLAT,), jnp.float32),
            pltpu.VMEM_SHARED((FLAT,), jnp.float32),
        ])
    out_flat = scatter_call(feats_r, depths_r, consts, sb_arr)
    out = out_flat.reshape(B, NPLANES, C, G, G)
    return out, sb


# speedup vs baseline: 1.3779x; 1.0032x over previous
import jax
import jax.numpy as jnp
from jax import lax
from jax.experimental import pallas as pl
from jax.experimental.pallas import tpu as pltpu
from jax.experimental.pallas import tpu_sc as plsc

G = 128
G2 = G * G
NPLANES = 3
FLAT = NPLANES * G2
DUMP = FLAT
CHB = 1024
CHM = 1024
L = 16
HW = 65536
NV = 4
NC = 96
NCHUNK = HW // CHB
NMCH = FLAT // CHM


def _read_consts(cbuf):
    va = cbuf[pl.ds(0, 16)]
    vb = cbuf[pl.ds(16, 16)]
    vals = [va[i] for i in range(16)] + [vb[i] for i in range(8)]
    return vals[0:9], vals[9:18], vals[18:21]


def _bf16_round(x):
    bits = plsc.bitcast(x, jnp.int32)
    rounded = (bits + 0x8000 + ((bits >> 16) & 1)) & jnp.int32(-65536)
    return plsc.bitcast(rounded, jnp.float32)


def _project16(k, r, t, u, vr, dd):
    cx = k[0] * u + k[1] * vr + k[2]
    cy = k[3] * u + k[4] * vr + k[5]
    cz = k[6] * u + k[7] * vr + k[8]
    px = _bf16_round(cx * dd)
    py = _bf16_round(cy * dd)
    pz = _bf16_round(cz * dd)
    wx = r[0] * px + r[1] * py + r[2] * pz + t[0]
    wy = r[3] * px + r[4] * py + r[5] * pz + t[1]
    wz = r[6] * px + r[7] * py + r[8] * pz + t[2]
    return wx, wy, wz


def _bounds_kernel(depths_hbm, consts_hbm, out_hbm, fbuf, cbuf, vbuf):
    c = lax.axis_index("c")
    s = lax.axis_index("s")
    wid = c * 16 + s
    vi = wid // 4
    qoff = (wid % 4) * 16384

    pltpu.sync_copy(consts_hbm.at[pl.ds(vi * 32, 32)], cbuf)
    k, r, t = _read_consts(cbuf)
    lanes = lax.iota(jnp.int32, 16).astype(jnp.float32)

    inf = jnp.full((L,), jnp.inf, jnp.float32)
    for i in range(3):
        vbuf[pl.ds(i * 16, L)] = inf
        vbuf[pl.ds((3 + i) * 16, L)] = -inf

    def chunk(ci, _):
        off = qoff + ci * CHB
        pltpu.sync_copy(depths_hbm.at[pl.ds(vi * HW + off, CHB)],
                        fbuf.at[pl.ds(0, CHB)])
        row0 = off // 256

        def jbody(j, carry):
            mnx, mny, mnz, mxx, mxy, mxz = carry
            u0 = (j % 16) * 16
            vr = (row0 + j // 16).astype(jnp.float32)
            u = u0.astype(jnp.float32) + lanes
            d = fbuf[pl.ds(j * 16, L)]
            mask = d != 0.0
            dd = jnp.where(mask, d + 0.01, d)
            wx, wy, wz = _project16(k, r, t, u, vr, dd)
            mnx = jnp.minimum(mnx, jnp.where(mask, wx, jnp.inf))
            mny = jnp.minimum(mny, jnp.where(mask, wy, jnp.inf))
            mnz = jnp.minimum(mnz, jnp.where(mask, wz, jnp.inf))
            mxx = jnp.maximum(mxx, jnp.where(mask, wx, -jnp.inf))
            mxy = jnp.maximum(mxy, jnp.where(mask, wy, -jnp.inf))
            mxz = jnp.maximum(mxz, jnp.where(mask, wz, -jnp.inf))
            return mnx, mny, mnz, mxx, mxy, mxz

        init = tuple(vbuf[pl.ds(i * 16, L)] for i in range(6))
        res = lax.fori_loop(0, CHB // L, jbody, init)
        for i in range(6):
            vbuf[pl.ds(i * 16, L)] = res[i]
        return 0

    lax.fori_loop(0, 16384 // CHB, chunk, 0)
    pltpu.sync_copy(vbuf, out_hbm.at[pl.ds(wid * 96, 96)])


def _scatter_kernel(feats_hbm, depths_hbm, consts_hbm, sb_hbm, out_hbm,
                    sums, fbuf, ibuf, rbuf, obuf, tbuf, cbuf, sbuf,
                    sf0, sf1, si0, si1, sr0, sr1, so0, so1, st0, st1,
                    sw0, sw1, idx_sp, recip_sp, trec_sp):
    c = lax.axis_index("c")
    s = lax.axis_index("s")
    b = c
    lanes = lax.iota(jnp.int32, 16).astype(jnp.float32)
    zero16 = jnp.zeros((L,), jnp.float32)
    ones16 = jnp.ones((L,), jnp.float32)
    semf = (sf0, sf1)
    semi = (si0, si1)
    semr = (sr0, sr1)
    semo = (so0, so1)
    semt = (st0, st1)
    semw = (sw0, sw1)

    v1 = s // 4
    qoff = (s % 4) * 16384
    pltpu.sync_copy(sb_hbm, sbuf)
    sbv = sbuf[pl.ds(0, 16)]
    sb = [sbv[i] for i in range(6)]
    den_x = sb[1] - sb[0]
    den_y = sb[3] - sb[2]
    den_z = sb[5] - sb[4]
    pltpu.sync_copy(consts_hbm.at[pl.ds((b * NV + v1) * 32, 32)], cbuf)
    k, r, t = _read_consts(cbuf)

    def b1_chunk(ci, _):
        off = qoff + ci * CHB
        pltpu.sync_copy(depths_hbm.at[pl.ds((b * NV + v1) * HW + off, CHB)],
                        fbuf.at[pl.ds(0, CHB)])
        row0 = off // 256

        def jbody(j, _):
            u0 = (j % 16) * 16
            vr = (row0 + j // 16).astype(jnp.float32)
            u = u0.astype(jnp.float32) + lanes
            d = fbuf[pl.ds(j * 16, L)]
            mask = d != 0.0
            dd = jnp.where(mask, d + 0.02, d)
            wx, wy, wz = _project16(k, r, t, u, vr, dd)
            nx = 2.0 * (wx - sb[0]) / den_x - 1.0
            ny = 2.0 * (wy - sb[2]) / den_y - 1.0
            nz = 2.0 * (wz - sb[4]) / den_z - 1.0
            cxi = jnp.clip(((nx * 0.5 + 0.5) * (G - 1)).astype(jnp.int32), 0, G - 1)
            cyi = jnp.clip(((ny * 0.5 + 0.5) * (G - 1)).astype(jnp.int32), 0, G - 1)
            czi = jnp.clip(((nz * 0.5 + 0.5) * (G - 1)).astype(jnp.int32), 0, G - 1)
            dump = jnp.full((L,), DUMP, jnp.int32)
            ibuf[pl.ds(j * 16, L)] = jnp.where(mask, cxi * G + cyi, dump)
            ibuf[pl.ds(CHB + j * 16, L)] = jnp.where(
                mask, cxi * G + czi + G2, dump)
            ibuf[pl.ds(2 * CHB + j * 16, L)] = jnp.where(
                mask, cyi * G + czi + 2 * G2, dump)
            return 0

        lax.fori_loop(0, CHB // L, jbody, 0)
        g = qoff // CHB + ci
        pltpu.sync_copy(ibuf.at[pl.ds(0, 3 * CHB)],
                        idx_sp.at[pl.ds(v1 * 3 * HW + g * 3 * CHB, 3 * CHB)])
        return 0

    lax.fori_loop(0, 16384 // CHB, b1_chunk, 0)
    plsc.subcore_barrier()

    @pl.when(s < 4)
    def _b2():
        v = s

        def zbody(i, _):
            sums[pl.ds(i * 16, L)] = zero16
            return 0

        lax.fori_loop(0, (FLAT + L) // L, zbody, 0)

        def cchunk(gi, _):
            pltpu.sync_copy(
                idx_sp.at[pl.ds(v * 3 * HW + gi * 3 * CHB, 3 * CHB)],
                ibuf.at[pl.ds(0, 3 * CHB)])

            def jbody(j, _):
                for p in range(NPLANES):
                    iv = ibuf[pl.ds(p * CHB + j * 16, L)]
                    plsc.addupdate_scatter(sums, [iv], ones16)
                return 0

            lax.fori_loop(0, CHB // L, jbody, 0)
            return 0

        lax.fori_loop(0, NCHUNK, cchunk, 0)

        def rbody(i, _):
            cnt = sums[pl.ds(i * 16, L)]
            sums[pl.ds(i * 16, L)] = jnp.where(cnt > 0.0, 1.0 / cnt, 0.0)
            return 0

        lax.fori_loop(0, FLAT // L, rbody, 0)
        pltpu.sync_copy(sums.at[pl.ds(0, FLAT)],
                        recip_sp.at[pl.ds(v * FLAT, FLAT)])

    plsc.subcore_barrier()

    span = 1024
    for sub in range(3):
        base = s * 3072 + sub * span

        def ztbody(i, _):
            tbuf[pl.ds(i * 16, L)] = zero16
            return 0

        lax.fori_loop(0, span // L, ztbody, 0)
        for v in range(NV):
            pltpu.sync_copy(recip_sp.at[pl.ds(v * FLAT + base, span)],
                            rbuf.at[pl.ds(0, span)])

            def tbody(i, _):
                rv = rbuf[pl.ds(i * 16, L)]
                tbuf[pl.ds(i * 16, L)] += jnp.where(rv > 0.0, 1.0, 0.0)
                return 0

            lax.fori_loop(0, span // L, tbody, 0)

        def trbody(i, _):
            tv = tbuf[pl.ds(i * 16, L)]
            tbuf[pl.ds(i * 16, L)] = 1.0 / jnp.maximum(tv, 1e-6)
            return 0

        lax.fori_loop(0, span // L, trbody, 0)
        pltpu.sync_copy(tbuf.at[pl.ds(0, span)], trec_sp.at[pl.ds(base, span)])
    plsc.subcore_barrier()

    def b4_channel(kch, _):
        ch = kch * 16 + s

        for v in range(NV):
            fbase = ((b * NV + v) * NC + ch) * HW
            ibase = v * 3 * HW

            def zsbody(i, _):
                sums[pl.ds(i * 16, L)] = zero16
                return 0

            lax.fori_loop(0, (FLAT + L) // L, zsbody, 0)

            def sc_cp(g, sl):
                return (
                    pltpu.make_async_copy(
                        feats_hbm.at[pl.ds(fbase + g * CHB, CHB)],
                        fbuf.at[pl.ds(sl * CHB, CHB)], semf[sl]),
                    pltpu.make_async_copy(
                        idx_sp.at[pl.ds(ibase + g * 3 * CHB, 3 * CHB)],
                        ibuf.at[pl.ds(sl * 3 * CHB, 3 * CHB)], semi[sl]),
                )

            def sc_start(g, sl):
                for cp in sc_cp(g, sl):
                    cp.start()

            def sc_wait(g, sl):
                for cp in sc_cp(g, sl):
                    cp.wait()

            def sc_compute(sl):
                def jbody(j, _):
                    fvs = [fbuf[pl.ds(sl * CHB + (4 * j + u) * 16, L)]
                           for u in range(4)]
                    for p in range(NPLANES):
                        for u in range(4):
                            iv = ibuf[pl.ds(sl * 3 * CHB + p * CHB
                                            + (4 * j + u) * 16, L)]
                            plsc.addupdate_scatter(sums, [iv], fvs[u])
                    return 0

                lax.fori_loop(0, CHB // L // 4, jbody, 0)

            sc_start(0, 0)

            def sc_pair(ci, _):
                g0 = ci * 2
                g1 = g0 + 1
                gn = jnp.minimum(g0 + 2, NCHUNK - 1)
                sc_start(g1, 1)
                sc_wait(g0, 0)
                sc_compute(0)
                sc_start(gn, 0)
                sc_wait(g1, 1)
                sc_compute(1)
                return 0

            lax.fori_loop(0, NCHUNK // 2, sc_pair, 0)
            sc_wait(NCHUNK - 1, 0)

            def m_ooff(g):
                return ((b * NPLANES + g // 16) * NC + ch) * G2 \
                    + (g % 16) * CHM

            def m_reads(g, sl):
                cps = [pltpu.make_async_copy(
                    recip_sp.at[pl.ds(v * FLAT + g * CHM, CHM)],
                    rbuf.at[pl.ds(sl * CHM, CHM)], semr[sl])]
                if v > 0:
                    cps.append(pltpu.make_async_copy(
                        out_hbm.at[pl.ds(m_ooff(g), CHM)],
                        obuf.at[pl.ds(sl * CHM, CHM)], semo[sl]))
                if v == NV - 1:
                    cps.append(pltpu.make_async_copy(
                        trec_sp.at[pl.ds(g * CHM, CHM)],
                        tbuf.at[pl.ds(sl * CHM, CHM)], semt[sl]))
                return cps

            def m_write(g, sl):
                return pltpu.make_async_copy(
                    obuf.at[pl.ds(sl * CHM, CHM)],
                    out_hbm.at[pl.ds(m_ooff(g), CHM)], semw[sl])

            def m_compute(g, sl):
                def jbody(j, _):
                    sl16 = pl.ds(sl * CHM + j * 16, L)
                    mean = sums[pl.ds(g * CHM + j * 16, L)] * rbuf[sl16]
                    if v == 0:
                        acc = mean
                    else:
                        acc = obuf[sl16] + mean
                    if v == NV - 1:
                        acc = acc * tbuf[sl16]
                    obuf[sl16] = acc
                    return 0

                lax.fori_loop(0, CHM // L, jbody, 0)

            def m_pair(ci, _):
                g0 = ci * 2
                g1 = g0 + 1

                @pl.when(ci > 0)
                def _drain():
                    m_write(g0, 0).wait()
                    m_write(g1, 1).wait()

                for cp in m_reads(g0, 0):
                    cp.start()
                for cp in m_reads(g1, 1):
                    cp.start()
                for cp in m_reads(g0, 0):
                    cp.wait()
                m_compute(g0, 0)
                m_write(g0, 0).start()
                for cp in m_reads(g1, 1):
                    cp.wait()
                m_compute(g1, 1)
                m_write(g1, 1).start()
                return 0

            lax.fori_loop(0, NMCH // 2, m_pair, 0)
            m_write(NMCH - 2, 0).wait()
            m_write(NMCH - 1, 1).wait()
        return 0

    lax.fori_loop(0, NC // 16, b4_channel, 0)


def kernel(image_features, depths, c2w_cond, intrinsic_cond):
    B, Nv, C, H, W = image_features.shape
    feats_r = image_features.reshape(-1)
    depths_r = depths.reshape(-1)

    k_inv = jnp.linalg.inv(intrinsic_cond)
    rot = c2w_cond[:, :, :3, :3]
    trans = c2w_cond[:, :, :3, 3]
    consts = jnp.concatenate(
        [k_inv.reshape(B, Nv, 9), rot.reshape(B, Nv, 9), trans,
         jnp.zeros((B, Nv, 11), jnp.float32)], axis=-1).reshape(-1)
    consts = consts.astype(jnp.bfloat16).astype(jnp.float32)

    mesh = plsc.VectorSubcoreMesh(core_axis_name="c", subcore_axis_name="s",
                                  num_cores=2, num_subcores=16)

    bounds_call = pl.kernel(
        _bounds_kernel, mesh=mesh,
        compiler_params=pltpu.CompilerParams(needs_layout_passes=False),
        out_type=jax.ShapeDtypeStruct((32 * 96,), jnp.float32),
        scratch_types=[
            pltpu.VMEM((CHB,), jnp.float32),
            pltpu.VMEM((32,), jnp.float32),
            pltpu.VMEM((96,), jnp.float32),
        ])
    parts = bounds_call(depths_r, consts).reshape(32, 6, 16)

    mins = jnp.minimum(parts[:, :3].min(axis=(0, 2)), 0.0)
    maxs = jnp.maximum(parts[:, 3:].max(axis=(0, 2)), 0.0)
    b0, b2_, b4 = mins[0], mins[1], mins[2]
    b1, b3, b5 = maxs[0], maxs[1], maxs[2]
    pad = 0.05
    sb = (b0 - pad * (b1 - b0), b1 + pad * (b1 - b0),
          b2_ - pad * (b3 - b2_), b3 + pad * (b3 - b2_),
          b4 - pad * (b5 - b4), b5 + pad * (b5 - b4))
    sb_arr = jnp.concatenate([jnp.stack(sb), jnp.zeros((10,), jnp.float32)])

    scatter_call = pl.kernel(
        _scatter_kernel, mesh=mesh,
        compiler_params=pltpu.CompilerParams(needs_layout_passes=False),
        out_type=jax.ShapeDtypeStruct((B * NPLANES * C * G2,), jnp.float32),
        scratch_types=[
            pltpu.VMEM((FLAT + L,), jnp.float32),
            pltpu.VMEM((2 * CHB,), jnp.float32),
            pltpu.VMEM((2 * 3 * CHB,), jnp.int32),
            pltpu.VMEM((2 * CHM,), jnp.float32),
            pltpu.VMEM((2 * CHM,), jnp.float32),
            pltpu.VMEM((2 * CHM,), jnp.float32),
            pltpu.VMEM((32,), jnp.float32),
            pltpu.VMEM((16,), jnp.float32),
        ] + [pltpu.SemaphoreType.DMA] * 12 + [
            pltpu.VMEM_SHARED((NV * NPLANES * HW,), jnp.int32),
            pltpu.VMEM_SHARED((NV * FLAT,), jnp.float32),
            pltpu.VMEM_SHARED((FLAT,), jnp.float32),
        ])
    out_flat = scatter_call(feats_r, depths_r, consts, sb_arr)
    out = out_flat.reshape(B, NPLANES, C, G, G)
    return out, sb


# speedup vs baseline: 1.4820x; 1.0756x over previous
import jax
import jax.numpy as jnp
from jax import lax
from jax.experimental import pallas as pl
from jax.experimental.pallas import tpu as pltpu
from jax.experimental.pallas import tpu_sc as plsc

G = 128
G2 = G * G
NPLANES = 3
FLAT = NPLANES * G2
DUMP = FLAT
CHB = 1024
CHM = 1024
L = 16
HW = 65536
NV = 4
NC = 96
NCHUNK = HW // CHB
NMCH = FLAT // CHM


def _read_consts(cbuf):
    va = cbuf[pl.ds(0, 16)]
    vb = cbuf[pl.ds(16, 16)]
    vals = [va[i] for i in range(16)] + [vb[i] for i in range(8)]
    return vals[0:9], vals[9:18], vals[18:21]


def _bf16_round(x):
    bits = plsc.bitcast(x, jnp.int32)
    rounded = (bits + 0x8000 + ((bits >> 16) & 1)) & jnp.int32(-65536)
    return plsc.bitcast(rounded, jnp.float32)


def _project16(k, r, t, u, vr, dd):
    cx = k[0] * u + k[1] * vr + k[2]
    cy = k[3] * u + k[4] * vr + k[5]
    cz = k[6] * u + k[7] * vr + k[8]
    px = _bf16_round(cx * dd)
    py = _bf16_round(cy * dd)
    pz = _bf16_round(cz * dd)
    wx = r[0] * px + r[1] * py + r[2] * pz + t[0]
    wy = r[3] * px + r[4] * py + r[5] * pz + t[1]
    wz = r[6] * px + r[7] * py + r[8] * pz + t[2]
    return wx, wy, wz


def _bounds_kernel(depths_hbm, consts_hbm, out_hbm, fbuf, cbuf, vbuf):
    c = lax.axis_index("c")
    s = lax.axis_index("s")
    wid = c * 16 + s
    vi = wid // 4
    qoff = (wid % 4) * 16384

    pltpu.sync_copy(consts_hbm.at[pl.ds(vi * 32, 32)], cbuf)
    k, r, t = _read_consts(cbuf)
    lanes = lax.iota(jnp.int32, 16).astype(jnp.float32)

    inf = jnp.full((L,), jnp.inf, jnp.float32)
    for i in range(3):
        vbuf[pl.ds(i * 16, L)] = inf
        vbuf[pl.ds((3 + i) * 16, L)] = -inf

    def chunk(ci, _):
        off = qoff + ci * CHB
        pltpu.sync_copy(depths_hbm.at[pl.ds(vi * HW + off, CHB)],
                        fbuf.at[pl.ds(0, CHB)])
        row0 = off // 256

        def jbody(j, carry):
            mnx, mny, mnz, mxx, mxy, mxz = carry
            u0 = (j % 16) * 16
            vr = (row0 + j // 16).astype(jnp.float32)
            u = u0.astype(jnp.float32) + lanes
            d = fbuf[pl.ds(j * 16, L)]
            mask = d != 0.0
            dd = jnp.where(mask, d + 0.01, d)
            wx, wy, wz = _project16(k, r, t, u, vr, dd)
            mnx = jnp.minimum(mnx, jnp.where(mask, wx, jnp.inf))
            mny = jnp.minimum(mny, jnp.where(mask, wy, jnp.inf))
            mnz = jnp.minimum(mnz, jnp.where(mask, wz, jnp.inf))
            mxx = jnp.maximum(mxx, jnp.where(mask, wx, -jnp.inf))
            mxy = jnp.maximum(mxy, jnp.where(mask, wy, -jnp.inf))
            mxz = jnp.maximum(mxz, jnp.where(mask, wz, -jnp.inf))
            return mnx, mny, mnz, mxx, mxy, mxz

        init = tuple(vbuf[pl.ds(i * 16, L)] for i in range(6))
        res = lax.fori_loop(0, CHB // L, jbody, init)
        for i in range(6):
            vbuf[pl.ds(i * 16, L)] = res[i]
        return 0

    lax.fori_loop(0, 16384 // CHB, chunk, 0)
    pltpu.sync_copy(vbuf, out_hbm.at[pl.ds(wid * 96, 96)])


def _scatter_kernel(feats_hbm, depths_hbm, consts_hbm, sb_hbm, out_hbm,
                    sums, fbuf, ibuf, rbuf, obuf, tbuf, cbuf, sbuf,
                    sf0, sf1, si0, si1, sr0, sr1, so0, so1, st0, st1,
                    sw0, sw1, idx_sp, recip_sp, trec_sp):
    c = lax.axis_index("c")
    s = lax.axis_index("s")
    b = c
    lanes = lax.iota(jnp.int32, 16).astype(jnp.float32)
    zero16 = jnp.zeros((L,), jnp.float32)
    ones16 = jnp.ones((L,), jnp.float32)
    semf = (sf0, sf1)
    semi = (si0, si1)
    semr = (sr0, sr1)
    semo = (so0, so1)
    semt = (st0, st1)
    semw = (sw0, sw1)

    v1 = s // 4
    qoff = (s % 4) * 16384
    pltpu.sync_copy(sb_hbm, sbuf)
    sbv = sbuf[pl.ds(0, 16)]
    sb = [sbv[i] for i in range(6)]
    den_x = sb[1] - sb[0]
    den_y = sb[3] - sb[2]
    den_z = sb[5] - sb[4]
    pltpu.sync_copy(consts_hbm.at[pl.ds((b * NV + v1) * 32, 32)], cbuf)
    k, r, t = _read_consts(cbuf)

    def b1_chunk(ci, _):
        off = qoff + ci * CHB
        pltpu.sync_copy(depths_hbm.at[pl.ds((b * NV + v1) * HW + off, CHB)],
                        fbuf.at[pl.ds(0, CHB)])
        row0 = off // 256

        def jbody(j, _):
            u0 = (j % 16) * 16
            vr = (row0 + j // 16).astype(jnp.float32)
            u = u0.astype(jnp.float32) + lanes
            d = fbuf[pl.ds(j * 16, L)]
            mask = d != 0.0
            dd = jnp.where(mask, d + 0.02, d)
            wx, wy, wz = _project16(k, r, t, u, vr, dd)
            nx = 2.0 * (wx - sb[0]) / den_x - 1.0
            ny = 2.0 * (wy - sb[2]) / den_y - 1.0
            nz = 2.0 * (wz - sb[4]) / den_z - 1.0
            cxi = jnp.clip(((nx * 0.5 + 0.5) * (G - 1)).astype(jnp.int32), 0, G - 1)
            cyi = jnp.clip(((ny * 0.5 + 0.5) * (G - 1)).astype(jnp.int32), 0, G - 1)
            czi = jnp.clip(((nz * 0.5 + 0.5) * (G - 1)).astype(jnp.int32), 0, G - 1)
            dump = jnp.full((L,), DUMP, jnp.int32)
            ibuf[pl.ds(j * 16, L)] = jnp.where(mask, cxi * G + cyi, dump)
            ibuf[pl.ds(CHB + j * 16, L)] = jnp.where(
                mask, cxi * G + czi + G2, dump)
            ibuf[pl.ds(2 * CHB + j * 16, L)] = jnp.where(
                mask, cyi * G + czi + 2 * G2, dump)
            return 0

        lax.fori_loop(0, CHB // L, jbody, 0)
        g = qoff // CHB + ci
        pltpu.sync_copy(ibuf.at[pl.ds(0, 3 * CHB)],
                        idx_sp.at[pl.ds(v1 * 3 * HW + g * 3 * CHB, 3 * CHB)])
        return 0

    lax.fori_loop(0, 16384 // CHB, b1_chunk, 0)
    plsc.subcore_barrier()

    @pl.when(s < 4)
    def _b2():
        v = s

        def zbody(i, _):
            sums[pl.ds(i * 16, L)] = zero16
            return 0

        lax.fori_loop(0, (FLAT + L) // L, zbody, 0)

        def cchunk(gi, _):
            pltpu.sync_copy(
                idx_sp.at[pl.ds(v * 3 * HW + gi * 3 * CHB, 3 * CHB)],
                ibuf.at[pl.ds(0, 3 * CHB)])

            def jbody(j, _):
                for p in range(NPLANES):
                    iv = ibuf[pl.ds(p * CHB + j * 16, L)]
                    plsc.addupdate_scatter(sums, [iv], ones16)
                return 0

            lax.fori_loop(0, CHB // L, jbody, 0)
            return 0

        lax.fori_loop(0, NCHUNK, cchunk, 0)

        def rbody(i, _):
            cnt = sums[pl.ds(i * 16, L)]
            sums[pl.ds(i * 16, L)] = jnp.where(cnt > 0.0, 1.0 / cnt, 0.0)
            return 0

        lax.fori_loop(0, FLAT // L, rbody, 0)
        pltpu.sync_copy(sums.at[pl.ds(0, FLAT)],
                        recip_sp.at[pl.ds(v * FLAT, FLAT)])

    plsc.subcore_barrier()

    span = 1024
    for sub in range(3):
        base = s * 3072 + sub * span

        def ztbody(i, _):
            tbuf[pl.ds(i * 16, L)] = zero16
            return 0

        lax.fori_loop(0, span // L, ztbody, 0)
        for v in range(NV):
            pltpu.sync_copy(recip_sp.at[pl.ds(v * FLAT + base, span)],
                            rbuf.at[pl.ds(0, span)])

            def tbody(i, _):
                rv = rbuf[pl.ds(i * 16, L)]
                tbuf[pl.ds(i * 16, L)] += jnp.where(rv > 0.0, 1.0, 0.0)
                return 0

            lax.fori_loop(0, span // L, tbody, 0)

        def trbody(i, _):
            tv = tbuf[pl.ds(i * 16, L)]
            tbuf[pl.ds(i * 16, L)] = 1.0 / jnp.maximum(tv, 1e-6)
            return 0

        lax.fori_loop(0, span // L, trbody, 0)
        pltpu.sync_copy(tbuf.at[pl.ds(0, span)], trec_sp.at[pl.ds(base, span)])
    plsc.subcore_barrier()

    def b4_channel(kch, _):
        ch = kch * 16 + s

        for v in range(NV):
            fbase = ((b * NV + v) * NC + ch) * HW
            ibase = v * 3 * HW

            def zsbody(i, _):
                for u in range(8):
                    sums[pl.ds((8 * i + u) * 16, L)] = zero16
                return 0

            lax.fori_loop(0, (FLAT + L) // L // 8, zsbody, 0)
            sums[pl.ds(FLAT, L)] = zero16

            def sc_cp(g, sl):
                return (
                    pltpu.make_async_copy(
                        feats_hbm.at[pl.ds(fbase + g * CHB, CHB)],
                        fbuf.at[pl.ds(sl * CHB, CHB)], semf[sl]),
                    pltpu.make_async_copy(
                        idx_sp.at[pl.ds(ibase + g * 3 * CHB, 3 * CHB)],
                        ibuf.at[pl.ds(sl * 3 * CHB, 3 * CHB)], semi[sl]),
                )

            def sc_start(g, sl):
                for cp in sc_cp(g, sl):
                    cp.start()

            def sc_wait(g, sl):
                for cp in sc_cp(g, sl):
                    cp.wait()

            def sc_compute(sl):
                def jbody(j, _):
                    fvs = [fbuf[pl.ds(sl * CHB + (4 * j + u) * 16, L)]
                           for u in range(4)]
                    for p in range(NPLANES):
                        for u in range(4):
                            iv = ibuf[pl.ds(sl * 3 * CHB + p * CHB
                                            + (4 * j + u) * 16, L)]
                            plsc.addupdate_scatter(sums, [iv], fvs[u])
                    return 0

                lax.fori_loop(0, CHB // L // 4, jbody, 0)

            sc_start(0, 0)

            def sc_pair(ci, _):
                g0 = ci * 2
                g1 = g0 + 1
                gn = jnp.minimum(g0 + 2, NCHUNK - 1)
                sc_start(g1, 1)
                sc_wait(g0, 0)
                sc_compute(0)
                sc_start(gn, 0)
                sc_wait(g1, 1)
                sc_compute(1)
                return 0

            lax.fori_loop(0, NCHUNK // 2, sc_pair, 0)
            sc_wait(NCHUNK - 1, 0)

            def m_ooff(g):
                return ((b * NPLANES + g // 16) * NC + ch) * G2 \
                    + (g % 16) * CHM

            def m_reads(g, sl):
                cps = [pltpu.make_async_copy(
                    recip_sp.at[pl.ds(v * FLAT + g * CHM, CHM)],
                    rbuf.at[pl.ds(sl * CHM, CHM)], semr[sl])]
                if v > 0:
                    cps.append(pltpu.make_async_copy(
                        out_hbm.at[pl.ds(m_ooff(g), CHM)],
                        obuf.at[pl.ds(sl * CHM, CHM)], semo[sl]))
                if v == NV - 1:
                    cps.append(pltpu.make_async_copy(
                        trec_sp.at[pl.ds(g * CHM, CHM)],
                        tbuf.at[pl.ds(sl * CHM, CHM)], semt[sl]))
                return cps

            def m_write(g, sl):
                return pltpu.make_async_copy(
                    obuf.at[pl.ds(sl * CHM, CHM)],
                    out_hbm.at[pl.ds(m_ooff(g), CHM)], semw[sl])

            def m_compute(g, sl):
                def jbody(j, _):
                    for u in range(4):
                        sl16 = pl.ds(sl * CHM + (4 * j + u) * 16, L)
                        mean = (sums[pl.ds(g * CHM + (4 * j + u) * 16, L)]
                                * rbuf[sl16])
                        if v == 0:
                            acc = mean
                        else:
                            acc = obuf[sl16] + mean
                        if v == NV - 1:
                            acc = acc * tbuf[sl16]
                        obuf[sl16] = acc
                    return 0

                lax.fori_loop(0, CHM // L // 4, jbody, 0)

            def m_pair(ci, _):
                g0 = ci * 2
                g1 = g0 + 1

                @pl.when(ci > 0)
                def _drain():
                    m_write(g0, 0).wait()
                    m_write(g1, 1).wait()

                for cp in m_reads(g0, 0):
                    cp.start()
                for cp in m_reads(g1, 1):
                    cp.start()
                for cp in m_reads(g0, 0):
                    cp.wait()
                m_compute(g0, 0)
                m_write(g0, 0).start()
                for cp in m_reads(g1, 1):
                    cp.wait()
                m_compute(g1, 1)
                m_write(g1, 1).start()
                return 0

            lax.fori_loop(0, NMCH // 2, m_pair, 0)
            m_write(NMCH - 2, 0).wait()
            m_write(NMCH - 1, 1).wait()
        return 0

    lax.fori_loop(0, NC // 16, b4_channel, 0)


def kernel(image_features, depths, c2w_cond, intrinsic_cond):
    B, Nv, C, H, W = image_features.shape
    feats_r = image_features.reshape(-1)
    depths_r = depths.reshape(-1)

    k_inv = jnp.linalg.inv(intrinsic_cond)
    rot = c2w_cond[:, :, :3, :3]
    trans = c2w_cond[:, :, :3, 3]
    consts = jnp.concatenate(
        [k_inv.reshape(B, Nv, 9), rot.reshape(B, Nv, 9), trans,
         jnp.zeros((B, Nv, 11), jnp.float32)], axis=-1).reshape(-1)
    consts = consts.astype(jnp.bfloat16).astype(jnp.float32)

    mesh = plsc.VectorSubcoreMesh(core_axis_name="c", subcore_axis_name="s",
                                  num_cores=2, num_subcores=16)

    bounds_call = pl.kernel(
        _bounds_kernel, mesh=mesh,
        compiler_params=pltpu.CompilerParams(needs_layout_passes=False),
        out_type=jax.ShapeDtypeStruct((32 * 96,), jnp.float32),
        scratch_types=[
            pltpu.VMEM((CHB,), jnp.float32),
            pltpu.VMEM((32,), jnp.float32),
            pltpu.VMEM((96,), jnp.float32),
        ])
    parts = bounds_call(depths_r, consts).reshape(32, 6, 16)

    mins = jnp.minimum(parts[:, :3].min(axis=(0, 2)), 0.0)
    maxs = jnp.maximum(parts[:, 3:].max(axis=(0, 2)), 0.0)
    b0, b2_, b4 = mins[0], mins[1], mins[2]
    b1, b3, b5 = maxs[0], maxs[1], maxs[2]
    pad = 0.05
    sb = (b0 - pad * (b1 - b0), b1 + pad * (b1 - b0),
          b2_ - pad * (b3 - b2_), b3 + pad * (b3 - b2_),
          b4 - pad * (b5 - b4), b5 + pad * (b5 - b4))
    sb_arr = jnp.concatenate([jnp.stack(sb), jnp.zeros((10,), jnp.float32)])

    scatter_call = pl.kernel(
        _scatter_kernel, mesh=mesh,
        compiler_params=pltpu.CompilerParams(needs_layout_passes=False),
        out_type=jax.ShapeDtypeStruct((B * NPLANES * C * G2,), jnp.float32),
        scratch_types=[
            pltpu.VMEM((FLAT + L,), jnp.float32),
            pltpu.VMEM((2 * CHB,), jnp.float32),
            pltpu.VMEM((2 * 3 * CHB,), jnp.int32),
            pltpu.VMEM((2 * CHM,), jnp.float32),
            pltpu.VMEM((2 * CHM,), jnp.float32),
            pltpu.VMEM((2 * CHM,), jnp.float32),
            pltpu.VMEM((32,), jnp.float32),
            pltpu.VMEM((16,), jnp.float32),
        ] + [pltpu.SemaphoreType.DMA] * 12 + [
            pltpu.VMEM_SHARED((NV * NPLANES * HW,), jnp.int32),
            pltpu.VMEM_SHARED((NV * FLAT,), jnp.float32),
            pltpu.VMEM_SHARED((FLAT,), jnp.float32),
        ])
    out_flat = scatter_call(feats_r, depths_r, consts, sb_arr)
    out = out_flat.reshape(B, NPLANES, C, G, G)
    return out, sb
